# baseline (speedup 1.0000x reference)
import functools

import jax
import jax.numpy as jnp
from jax import lax
from jax.experimental import pallas as pl
from jax.experimental.pallas import tpu as pltpu
from jax.experimental.pallas import tpu_sc as plsc

ROWS = 1024
COLS = 100000
LANES = 16
NUM_CORES = 2
NUM_SUBCORES = 16
NUM_WORKERS = NUM_CORES * NUM_SUBCORES
ROWS_PER_WORKER = ROWS // NUM_WORKERS
RG = 8
NGROUP = ROWS_PER_WORKER // RG
CHUNK = 6144
NFULL = COLS // CHUNK
TAIL = COLS - NFULL * CHUNK
NPAIR = NFULL // 2
NSTEP = CHUNK // LANES
NSTEP_T = TAIL // LANES
KAPPA = 1e30
NEG_INF = float("-inf")

_GATHER_DNUMS = lax.GatherDimensionNumbers(
    offset_dims=(), collapsed_slice_dims=(0,), start_index_map=(0,)
)


def _shuffle(v, idx):
    return lax.gather(
        v,
        idx.reshape(LANES, 1),
        _GATHER_DNUMS,
        slice_sizes=(1,),
        mode=lax.GatherScatterMode.PROMISE_IN_BOUNDS,
    )


def _butterfly(v, op, iota):
    for s in (8, 4, 2, 1):
        v = op(v, _shuffle(v, iota ^ s))
    return v


def _scan_block(buf, pairs, nstep):
    def step(j, pairs):
        col = j * LANES
        new = []
        for i, (m1, m2) in enumerate(pairs):
            v = buf[i, pl.ds(col, LANES)]
            t = jnp.minimum(m1, v)
            m1 = jnp.maximum(m1, v)
            m2 = jnp.maximum(m2, t)
            new.append((m1, m2))
        return tuple(new)

    return lax.fori_loop(0, nstep, step, pairs, unroll=2)


def _fresh_pairs():
    return tuple(
        (jnp.full((LANES,), NEG_INF, jnp.float32),
         jnp.full((LANES,), NEG_INF, jnp.float32))
        for _ in range(RG)
    )


def _label_vec(lab_buf, rl, iota):
    lblk = (rl // LANES) * LANES
    labv = lab_buf[pl.ds(lblk, LANES)].astype(jnp.float32)
    return _butterfly(
        jnp.where(iota == rl - lblk, labv, jnp.float32(-1.0)),
        jnp.maximum,
        iota,
    )


def _margin_body(logits_hbm, labels_hbm, out_hbm, buf0, buf1, buft, lab_buf,
                 corr_buf, out_buf, sem0, sem1, semt, semc):
    cid = lax.axis_index("c")
    sid = lax.axis_index("s")
    wid = sid * NUM_CORES + cid
    base = wid * ROWS_PER_WORKER

    pltpu.sync_copy(labels_hbm.at[pl.ds(base, ROWS_PER_WORKER)], lab_buf)

    iota = lax.iota(jnp.int32, LANES)

    lane_sel = []
    for rl in range(ROWS_PER_WORKER):
        label_fv = _label_vec(lab_buf, rl, iota)
        label_i = label_fv[0].astype(jnp.int32)
        lblk2 = (label_i // LANES) * LANES
        pltpu.async_copy(
            logits_hbm.at[base + rl].at[pl.ds(lblk2, LANES)],
            corr_buf.at[rl],
            semc,
        )
        lane_sel.append((label_fv, iota == label_i - lblk2))
    for rl in range(ROWS_PER_WORKER):
        pltpu.make_async_copy(
            logits_hbm.at[base + rl].at[pl.ds(0, LANES)],
            corr_buf.at[rl],
            semc,
        ).wait()

    def issue_chunk(row0, c, buf, sem):
        pltpu.async_copy(
            logits_hbm.at[pl.ds(row0, RG), pl.ds(c * CHUNK, CHUNK)],
            buf, sem)

    def wait_chunk(row0, c, buf, sem):
        pltpu.make_async_copy(
            logits_hbm.at[pl.ds(row0, RG), pl.ds(c * CHUNK, CHUNK)],
            buf, sem).wait()

    def issue_tail(row0, sem):
        pltpu.async_copy(
            logits_hbm.at[pl.ds(row0, RG), pl.ds(NFULL * CHUNK, TAIL)],
            buft, sem)

    def wait_tail(row0, sem):
        pltpu.make_async_copy(
            logits_hbm.at[pl.ds(row0, RG), pl.ds(NFULL * CHUNK, TAIL)],
            buft, sem).wait()

    out0 = jnp.zeros((LANES,), jnp.float32)
    out1 = jnp.zeros((LANES,), jnp.float32)

    issue_chunk(base, 0, buf0, sem0)
    for g in range(NGROUP):
        row0 = g * RG
        grow = base + row0

        def pair_step(c, pairs, grow=grow):
            wait_chunk(grow, 2 * c, buf0, sem0)
            issue_chunk(grow, 2 * c + 1, buf1, sem1)
            pairs = _scan_block(buf0, pairs, NSTEP)
            wait_chunk(grow, 2 * c + 1, buf1, sem1)

            @pl.when(c + 1 < NPAIR)
            def _():
                issue_chunk(grow, 2 * c + 2, buf0, sem0)

            @pl.when(c + 1 == NPAIR)
            def _():
                issue_tail(grow, semt)

            pairs = _scan_block(buf1, pairs, NSTEP)
            return pairs

        pairs = lax.fori_loop(0, NPAIR, pair_step, _fresh_pairs())

        wait_tail(grow, semt)
        if g + 1 < NGROUP:
            issue_chunk(grow + RG, 0, buf0, sem0)
        pairs = _scan_block(buft, pairs, NSTEP_T)

        for i in range(RG):
            rl = row0 + i
            m1, m2 = pairs[i]
            row_topv = _butterfly(m1, jnp.maximum, iota)
            eq = m1 == row_topv
            cntv = _butterfly(
                jnp.where(eq, jnp.float32(1.0), jnp.float32(0.0)),
                jnp.add, iota)
            m1_excl = jnp.where(eq, NEG_INF, m1)
            sec_m1 = _butterfly(m1_excl, jnp.maximum, iota)
            sec_m1 = jnp.where(cntv > 1.5, row_topv, sec_m1)
            row_secondv = jnp.maximum(
                sec_m1, _butterfly(m2, jnp.maximum, iota))

            label_fv, sel = lane_sel[rl]
            cv = corr_buf[rl, pl.ds(0, LANES)]
            correctv = _butterfly(
                jnp.where(sel, cv, NEG_INF), jnp.maximum, iota)

            max_incorrect = jnp.where(
                correctv == row_topv, row_secondv, row_topv)
            valv = jnp.minimum(max_incorrect - correctv, KAPPA)

            if rl < LANES:
                out0 = jnp.where(iota == rl, valv, out0)
            else:
                out1 = jnp.where(iota == rl - LANES, valv, out1)

    out_buf[pl.ds(0, LANES)] = out0
    out_buf[pl.ds(LANES, LANES)] = out1
    pltpu.sync_copy(out_buf, out_hbm.at[pl.ds(base, ROWS_PER_WORKER)])


from jax.experimental.layout import Format, Layout


def _margin_loss(logits, labels):
    mesh = plsc.VectorSubcoreMesh(core_axis_name="c", subcore_axis_name="s")
    fn = functools.partial(
        pl.kernel,
        mesh=mesh,
        out_type=jax.ShapeDtypeStruct((ROWS,), jnp.float32),
        scratch_types=[
            pltpu.VMEM((RG, CHUNK), jnp.float32),
            pltpu.VMEM((RG, CHUNK), jnp.float32),
            pltpu.VMEM((RG, TAIL), jnp.float32),
            pltpu.VMEM((ROWS_PER_WORKER,), jnp.int32),
            pltpu.VMEM((ROWS_PER_WORKER, LANES), jnp.float32),
            pltpu.VMEM((ROWS_PER_WORKER,), jnp.float32),
            pltpu.SemaphoreType.DMA,
            pltpu.SemaphoreType.DMA,
            pltpu.SemaphoreType.DMA,
            pltpu.SemaphoreType.DMA,
        ],
    )(_margin_body)
    return fn(logits, labels)


_JITTED = {}


def _get_jitted(lsh, labsh):
    key = (lsh, labsh)
    fn = _JITTED.get(key)
    if fn is None:
        fn = jax.jit(
            _margin_loss,
            in_shardings=(
                Format(Layout(major_to_minor=(0, 1)), lsh),
                Format(Layout(major_to_minor=(0,)), labsh),
            ),
        )
        _JITTED[key] = fn
    return fn


def kernel(logits, labels):
    labels = labels.astype(jnp.int32)
    try:
        fn = _get_jitted(logits.sharding, labels.sharding)
    except (AttributeError, ValueError):
        fn = jax.jit(_margin_loss)
    return fn(logits, labels)


# speedup vs baseline: 1.0002x; 1.0002x over previous
import functools

import jax
import jax.numpy as jnp
from jax import lax
from jax.experimental import pallas as pl
from jax.experimental.pallas import tpu as pltpu
from jax.experimental.pallas import tpu_sc as plsc

ROWS = 1024
COLS = 100000
LANES = 16
NUM_CORES = 2
NUM_SUBCORES = 16
NUM_WORKERS = NUM_CORES * NUM_SUBCORES
ROWS_PER_WORKER = ROWS // NUM_WORKERS
RG = 8
NGROUP = ROWS_PER_WORKER // RG
CHUNK = 6144
NFULL = COLS // CHUNK
TAIL = COLS - NFULL * CHUNK
NPAIR = NFULL // 2
NSTEP = CHUNK // LANES
NSTEP_T = TAIL // LANES
KAPPA = 1e30
NEG_INF = float("-inf")

_GATHER_DNUMS = lax.GatherDimensionNumbers(
    offset_dims=(), collapsed_slice_dims=(0,), start_index_map=(0,)
)


def _shuffle(v, idx):
    return lax.gather(
        v,
        idx.reshape(LANES, 1),
        _GATHER_DNUMS,
        slice_sizes=(1,),
        mode=lax.GatherScatterMode.PROMISE_IN_BOUNDS,
    )


def _butterfly(v, op, iota):
    for s in (8, 4, 2, 1):
        v = op(v, _shuffle(v, iota ^ s))
    return v


def _scan_block(buf, pairs, nstep):
    def step(j, pairs):
        col = j * LANES
        new = []
        for i, (m1, m2) in enumerate(pairs):
            v = buf[i, pl.ds(col, LANES)]
            t = jnp.minimum(m1, v)
            m1 = jnp.maximum(m1, v)
            m2 = jnp.maximum(m2, t)
            new.append((m1, m2))
        return tuple(new)

    return lax.fori_loop(0, nstep, step, pairs, unroll=2)


def _fresh_pairs():
    return tuple(
        (jnp.full((LANES,), NEG_INF, jnp.float32),
         jnp.full((LANES,), NEG_INF, jnp.float32))
        for _ in range(RG)
    )


def _label_vec(lab_buf, rl, iota):
    lblk = (rl // LANES) * LANES
    labv = lab_buf[pl.ds(lblk, LANES)].astype(jnp.float32)
    return _butterfly(
        jnp.where(iota == rl - lblk, labv, jnp.float32(-1.0)),
        jnp.maximum,
        iota,
    )


def _margin_body(logits_hbm, labels_hbm, out_hbm, buf0, buf1, buft, lab_buf,
                 corr_buf, out_buf, sem0, sem1, semt, semc):
    cid = lax.axis_index("c")
    sid = lax.axis_index("s")
    wid = sid * NUM_CORES + cid
    base = wid * ROWS_PER_WORKER

    pltpu.sync_copy(labels_hbm.at[pl.ds(base, ROWS_PER_WORKER)], lab_buf)

    iota = lax.iota(jnp.int32, LANES)

    lane_sel = []
    for rl in range(ROWS_PER_WORKER):
        label_fv = _label_vec(lab_buf, rl, iota)
        label_i = label_fv[0].astype(jnp.int32)
        lblk2 = (label_i // LANES) * LANES
        pltpu.async_copy(
            logits_hbm.at[base + rl].at[pl.ds(lblk2, LANES)],
            corr_buf.at[rl],
            semc,
        )
        lane_sel.append((label_fv, iota == label_i - lblk2))
    for rl in range(ROWS_PER_WORKER):
        pltpu.make_async_copy(
            logits_hbm.at[base + rl].at[pl.ds(0, LANES)],
            corr_buf.at[rl],
            semc,
        ).wait()

    def issue_chunk(row0, c, buf, sem):
        pltpu.async_copy(
            logits_hbm.at[pl.ds(row0, RG), pl.ds(c * CHUNK, CHUNK)],
            buf, sem)

    def wait_chunk(row0, c, buf, sem):
        pltpu.make_async_copy(
            logits_hbm.at[pl.ds(row0, RG), pl.ds(c * CHUNK, CHUNK)],
            buf, sem).wait()

    def issue_tail(row0, sem):
        pltpu.async_copy(
            logits_hbm.at[pl.ds(row0, RG), pl.ds(NFULL * CHUNK, TAIL)],
            buft, sem)

    def wait_tail(row0, sem):
        pltpu.make_async_copy(
            logits_hbm.at[pl.ds(row0, RG), pl.ds(NFULL * CHUNK, TAIL)],
            buft, sem).wait()

    out0 = jnp.zeros((LANES,), jnp.float32)
    out1 = jnp.zeros((LANES,), jnp.float32)

    issue_chunk(base, 0, buf0, sem0)
    for g in range(NGROUP):
        row0 = g * RG
        grow = base + row0

        def pair_step(c, pairs, grow=grow):
            wait_chunk(grow, 2 * c, buf0, sem0)
            issue_chunk(grow, 2 * c + 1, buf1, sem1)
            pairs = _scan_block(buf0, pairs, NSTEP)
            wait_chunk(grow, 2 * c + 1, buf1, sem1)

            @pl.when(c + 1 < NPAIR)
            def _():
                issue_chunk(grow, 2 * c + 2, buf0, sem0)

            @pl.when(c + 1 == NPAIR)
            def _():
                issue_tail(grow, semt)

            pairs = _scan_block(buf1, pairs, NSTEP)
            return pairs

        pairs = lax.fori_loop(0, NPAIR, pair_step, _fresh_pairs())

        wait_tail(grow, semt)
        if g + 1 < NGROUP:
            issue_chunk(grow + RG, 0, buf0, sem0)
        pairs = _scan_block(buft, pairs, NSTEP_T)

        for i in range(RG):
            rl = row0 + i
            m1, m2 = pairs[i]
            row_topv = _butterfly(m1, jnp.maximum, iota)
            eq = m1 == row_topv
            cntv = _butterfly(
                jnp.where(eq, jnp.float32(1.0), jnp.float32(0.0)),
                jnp.add, iota)
            m1_excl = jnp.where(eq, NEG_INF, m1)
            sec_m1 = _butterfly(m1_excl, jnp.maximum, iota)
            sec_m1 = jnp.where(cntv > 1.5, row_topv, sec_m1)
            row_secondv = jnp.maximum(
                sec_m1, _butterfly(m2, jnp.maximum, iota))

            label_fv, sel = lane_sel[rl]
            cv = corr_buf[rl, pl.ds(0, LANES)]
            correctv = _butterfly(
                jnp.where(sel, cv, NEG_INF), jnp.maximum, iota)

            max_incorrect = jnp.where(
                correctv == row_topv, row_secondv, row_topv)
            valv = jnp.minimum(max_incorrect - correctv, KAPPA)

            if rl < LANES:
                out0 = jnp.where(iota == rl, valv, out0)
            else:
                out1 = jnp.where(iota == rl - LANES, valv, out1)

    out_buf[pl.ds(0, LANES)] = out0
    out_buf[pl.ds(LANES, LANES)] = out1
    pltpu.sync_copy(out_buf, out_hbm.at[pl.ds(base, ROWS_PER_WORKER)])


from jax.experimental.layout import Format, Layout


def _margin_loss(logits, labels):
    mesh = plsc.VectorSubcoreMesh(core_axis_name="c", subcore_axis_name="s")
    fn = functools.partial(
        pl.kernel,
        mesh=mesh,
        compiler_params=pltpu.CompilerParams(use_tc_tiling_on_sc=True),
        out_type=jax.ShapeDtypeStruct((ROWS,), jnp.float32),
        scratch_types=[
            pltpu.VMEM((RG, CHUNK), jnp.float32),
            pltpu.VMEM((RG, CHUNK), jnp.float32),
            pltpu.VMEM((RG, TAIL), jnp.float32),
            pltpu.VMEM((ROWS_PER_WORKER,), jnp.int32),
            pltpu.VMEM((ROWS_PER_WORKER, LANES), jnp.float32),
            pltpu.VMEM((ROWS_PER_WORKER,), jnp.float32),
            pltpu.SemaphoreType.DMA,
            pltpu.SemaphoreType.DMA,
            pltpu.SemaphoreType.DMA,
            pltpu.SemaphoreType.DMA,
        ],
    )(_margin_body)
    return fn(logits, labels)


_JITTED = {}


def _get_jitted(lsh, labsh):
    key = (lsh, labsh)
    fn = _JITTED.get(key)
    if fn is None:
        fn = jax.jit(
            _margin_loss,
            in_shardings=(
                Format(Layout(major_to_minor=(0, 1)), lsh),
                Format(Layout(major_to_minor=(0,)), labsh),
            ),
        )
        _JITTED[key] = fn
    return fn


def kernel(logits, labels):
    labels = labels.astype(jnp.int32)
    try:
        fn = _get_jitted(logits.sharding, labels.sharding)
    except (AttributeError, ValueError):
        fn = jax.jit(_margin_loss)
    return fn(logits, labels)


# speedup vs baseline: 2.2355x; 2.2351x over previous
import functools

import jax
import jax.numpy as jnp
from jax import lax
from jax.experimental import pallas as pl
from jax.experimental.pallas import tpu as pltpu
from jax.experimental.pallas import tpu_sc as plsc

ROWS = 1024
COLS = 100000
LANES = 16
NUM_CORES = 2
NUM_SUBCORES = 16
NUM_WORKERS = NUM_CORES * NUM_SUBCORES
ROWS_PER_WORKER = ROWS // NUM_WORKERS
NBLK = ROWS // LANES
SLAB = 3120
VCHUNK = 48
NCHUNK = SLAB // VCHUNK
TAILV = COLS - NUM_WORKERS * SLAB
KAPPA = 1e30
NEG_INF = float("-inf")

_GATHER_DNUMS = lax.GatherDimensionNumbers(
    offset_dims=(), collapsed_slice_dims=(0,), start_index_map=(0,)
)


def _shuffle(v, idx):
    return lax.gather(
        v,
        idx.reshape(LANES, 1),
        _GATHER_DNUMS,
        slice_sizes=(1,),
        mode=lax.GatherScatterMode.PROMISE_IN_BOUNDS,
    )


def _bfly_max(v, iota):
    for s in (8, 4, 2, 1):
        v = jnp.maximum(v, _shuffle(v, iota ^ s))
    return v


def _scan_chunk(buf, state, nvocab):
    def qstep(q, _):
        for k in range(4):
            b = q * 4 + k
            m1 = state[pl.ds(b * 2 * LANES, LANES)]
            m2 = state[pl.ds(b * 2 * LANES + LANES, LANES)]
            for v in range(nvocab):
                x = buf[v, pl.ds(b * LANES, LANES)]
                t = jnp.minimum(m1, x)
                m1 = jnp.maximum(m1, x)
                m2 = jnp.maximum(m2, t)
            state[pl.ds(b * 2 * LANES, LANES)] = m1
            state[pl.ds(b * 2 * LANES + LANES, LANES)] = m2
        return 0

    lax.fori_loop(0, NBLK // 4, qstep, 0)


def _phase1_body(lt_hbm, part_hbm, buf0, buf1, buft, state, sem0, sem1):
    cid = lax.axis_index("c")
    sid = lax.axis_index("s")
    wid = sid * NUM_CORES + cid
    base_v = wid * SLAB

    ninf = jnp.full((LANES,), NEG_INF, jnp.float32)
    for b in range(NBLK):
        state[pl.ds(b * 2 * LANES, LANES)] = ninf
        state[pl.ds(b * 2 * LANES + LANES, LANES)] = ninf

    def issue(c, buf, sem):
        pltpu.async_copy(lt_hbm.at[pl.ds(base_v + c * VCHUNK, VCHUNK)],
                         buf, sem)

    def wait(c, buf, sem):
        pltpu.make_async_copy(
            lt_hbm.at[pl.ds(base_v + c * VCHUNK, VCHUNK)], buf, sem).wait()

    issue(0, buf0, sem0)

    def pair_step(c, _):
        wait(2 * c, buf0, sem0)
        issue(2 * c + 1, buf1, sem1)
        _scan_chunk(buf0, state, VCHUNK)
        wait(2 * c + 1, buf1, sem1)
        issue(2 * c + 2, buf0, sem0)
        _scan_chunk(buf1, state, VCHUNK)
        return 0

    lax.fori_loop(0, (NCHUNK - 1) // 2, pair_step, 0)
    wait(NCHUNK - 1, buf0, sem0)
    _scan_chunk(buf0, state, VCHUNK)

    @pl.when(wid == NUM_WORKERS - 1)
    def _():
        for k in range(TAILV // LANES):
            pltpu.sync_copy(
                lt_hbm.at[pl.ds(NUM_WORKERS * SLAB + k * LANES, LANES)],
                buft)
            _scan_chunk(buft, state, LANES)

    pltpu.sync_copy(state, part_hbm.at[wid])


def _phase2_body(lt_hbm, labels_hbm, part_hbm, out_hbm, pbuf, lab_buf,
                 corr_buf, out_buf, semc):
    cid = lax.axis_index("c")
    sid = lax.axis_index("s")
    wid = sid * NUM_CORES + cid
    rbase = wid * ROWS_PER_WORKER

    iota = lax.iota(jnp.int32, LANES)
    pltpu.sync_copy(labels_hbm.at[pl.ds(rbase, ROWS_PER_WORKER)], lab_buf)
    pltpu.sync_copy(part_hbm, pbuf)

    labels_f = []
    for rl in range(ROWS_PER_WORKER):
        lblk = (rl // LANES) * LANES
        labv = lab_buf[pl.ds(lblk, LANES)].astype(jnp.float32)
        lf = _bfly_max(
            jnp.where(iota == rl - lblk, labv, jnp.float32(-1.0)), iota)
        labels_f.append(lf)
        label_i = lf[0].astype(jnp.int32)
        rblk = rbase + lblk
        pltpu.async_copy(lt_hbm.at[label_i].at[pl.ds(rblk, LANES)],
                         corr_buf.at[rl], semc)
    for rl in range(ROWS_PER_WORKER):
        pltpu.make_async_copy(lt_hbm.at[0].at[pl.ds(0, LANES)],
                              corr_buf.at[rl], semc).wait()

    for half in range(2):
        b = 2 * wid + half
        m1 = pbuf[0, pl.ds(b * 2 * LANES, LANES)]
        m2 = pbuf[0, pl.ds(b * 2 * LANES + LANES, LANES)]
        for w in range(1, NUM_WORKERS):
            n1 = pbuf[w, pl.ds(b * 2 * LANES, LANES)]
            n2 = pbuf[w, pl.ds(b * 2 * LANES + LANES, LANES)]
            t = jnp.minimum(m1, n1)
            m1 = jnp.maximum(m1, n1)
            m2 = jnp.maximum(jnp.maximum(m2, n2), t)

        correct = jnp.full((LANES,), NEG_INF, jnp.float32)
        for l in range(LANES):
            rl = half * LANES + l
            cv = corr_buf[rl, pl.ds(0, LANES)]
            correct = jnp.where(iota == l, cv, correct)

        max_incorrect = jnp.where(correct == m1, m2, m1)
        valv = jnp.minimum(max_incorrect - correct, KAPPA)
        out_buf[pl.ds(half * LANES, LANES)] = valv

    pltpu.sync_copy(out_buf, out_hbm.at[pl.ds(rbase, ROWS_PER_WORKER)])


def _margin_loss(logits, labels):
    lt = logits.T
    mesh = plsc.VectorSubcoreMesh(core_axis_name="c", subcore_axis_name="s")
    p1 = functools.partial(
        pl.kernel,
        mesh=mesh,
        out_type=jax.ShapeDtypeStruct((NUM_WORKERS, 2 * ROWS), jnp.float32),
        scratch_types=[
            pltpu.VMEM((VCHUNK, ROWS), jnp.float32),
            pltpu.VMEM((VCHUNK, ROWS), jnp.float32),
            pltpu.VMEM((LANES, ROWS), jnp.float32),
            pltpu.VMEM((2 * ROWS,), jnp.float32),
            pltpu.SemaphoreType.DMA,
            pltpu.SemaphoreType.DMA,
        ],
    )(_phase1_body)
    partials = p1(lt)
    p2 = functools.partial(
        pl.kernel,
        mesh=mesh,
        out_type=jax.ShapeDtypeStruct((ROWS,), jnp.float32),
        scratch_types=[
            pltpu.VMEM((NUM_WORKERS, 2 * ROWS), jnp.float32),
            pltpu.VMEM((ROWS_PER_WORKER,), jnp.int32),
            pltpu.VMEM((ROWS_PER_WORKER, LANES), jnp.float32),
            pltpu.VMEM((ROWS_PER_WORKER,), jnp.float32),
            pltpu.SemaphoreType.DMA,
        ],
    )(_phase2_body)
    return p2(lt, labels, partials)


_JIT = jax.jit(_margin_loss)


def kernel(logits, labels):
    return _JIT(logits, labels.astype(jnp.int32))
